# baseline (speedup 1.0000x reference)
import functools

import jax
import jax.numpy as jnp
from jax import lax
from jax.experimental import pallas as pl
from jax.experimental.pallas import tpu as pltpu
from jax.experimental.pallas import tpu_sc as plsc

B = 16384
D = 50
DP = 128
DU = 64
NW = 32
ROWS_PER_W = B // NW
CHUNK = 128
NCHUNK = ROWS_PER_W // CHUNK
HALF = ROWS_PER_W // 2

BLK = 1024
F_FT = 768


def _sel(idx_v, kk):
    a = kk // CHUNK
    col = (kk % CHUNK) // 16 * 16
    iv = idx_v[a, pl.ds(col, 16)]
    lanes = lax.iota(jnp.int32, 16)
    dn = lax.GatherDimensionNumbers(
        offset_dims=(), collapsed_slice_dims=(0,), start_index_map=(0,))
    rot = jnp.bitwise_and(lanes + kk % 16, 15)
    sel = lax.gather(iv, rot.reshape(16, 1), dn, slice_sizes=(1,),
                     mode=lax.GatherScatterMode.PROMISE_IN_BOUNDS)
    return sel[0]


def _sc_gather_body(tab, idx_hbm, out, idx_v, g_v, rows_v, sem):
    c = lax.axis_index("c")
    s = lax.axis_index("s")
    wid = s * 2 + c
    base = wid * ROWS_PER_W
    pltpu.sync_copy(idx_hbm.at[pl.ds(pl.multiple_of(wid * 8, 8), 8)], idx_v)

    zeros16 = jnp.zeros((16,), jnp.float32)

    def zrow(r, _):
        for col in (50, 66, 82, 98, 112):
            rows_v[r, pl.ds(col, 16)] = zeros16
        return _

    lax.fori_loop(0, HALF, zrow, 0)

    for h in range(2):
        def issue(k, _, h=h):
            i = _sel(idx_v, h * HALF + k)
            pltpu.async_copy(tab.at[pl.ds(i, 1)], g_v.at[pl.ds(k, 1)], sem)
            return _

        lax.fori_loop(0, HALF, issue, 0)
        pltpu.make_async_copy(tab.at[pl.ds(0, HALF)], g_v, sem).wait()

        def repack(k, _):
            for o in (0, 16, 32, 34):
                rows_v[k, pl.ds(o, 16)] = g_v[k, pl.ds(o, 16)]
            return _

        lax.fori_loop(0, HALF, repack, 0)
        pltpu.sync_copy(
            rows_v,
            out.at[pl.ds(pl.multiple_of(base + h * HALF, 8), HALF)])


def _sc_gather_one(table, idx):
    mesh = plsc.VectorSubcoreMesh(core_axis_name="c", subcore_axis_name="s")
    return pl.kernel(
        _sc_gather_body,
        out_type=jax.ShapeDtypeStruct((B, DP), jnp.float32),
        mesh=mesh,
        scratch_types=[
            pltpu.VMEM((8, CHUNK), jnp.int32),
            pltpu.VMEM((HALF, D), jnp.float32),
            pltpu.VMEM((HALF, DP), jnp.float32),
            pltpu.SemaphoreType.DMA,
        ],
        compiler_params=pltpu.CompilerParams(use_tc_tiling_on_sc=True),
    )(table, idx)


def _sc_user_body(tab, idx_hbm, out, idx_v, sem):
    c = lax.axis_index("c")
    s = lax.axis_index("s")
    wid = s * 2 + c
    base = wid * ROWS_PER_W
    pltpu.sync_copy(idx_hbm.at[pl.ds(pl.multiple_of(wid * 8, 8), 8)], idx_v)

    def issue(k, _):
        i = _sel(idx_v, k)
        i16 = pl.multiple_of(jnp.bitwise_and(i, ~15), 16)
        pltpu.async_copy(
            tab.at[pl.ds(i16, 16)],
            out.at[pl.ds(pl.multiple_of((base + k) * 16, 16), 16)], sem)
        return _

    lax.fori_loop(0, ROWS_PER_W, issue, 0)
    pltpu.make_async_copy(
        tab.at[pl.ds(0, ROWS_PER_W * 16)],
        out.at[pl.ds(pl.multiple_of(base * 16, 16), ROWS_PER_W * 16)],
        sem).wait()


def _sc_gather_user(table16, idx):
    mesh = plsc.VectorSubcoreMesh(core_axis_name="c", subcore_axis_name="s")
    return pl.kernel(
        _sc_user_body,
        out_type=jax.ShapeDtypeStruct((B * 16, DU), jnp.bfloat16),
        mesh=mesh,
        scratch_types=[
            pltpu.VMEM((8, CHUNK), jnp.int32),
            pltpu.SemaphoreType.DMA,
        ],
        compiler_params=pltpu.CompilerParams(use_tc_tiling_on_sc=True),
    )(table16, idx)


def _mlp_body(ft, ub, oh, b_e, a_e, p_e, cat, scal,
              w1t, w1u, w1b, w1a, w1p, w1c, w1s, b1, w2, b2, w3, b3,
              out_ref):
    f32 = jnp.float32
    acc = jnp.dot(ft[...], w1t[...], preferred_element_type=f32)
    xb = ub[...].astype(f32).reshape(BLK, 16, DU)
    u_sel = jnp.sum(xb * oh[...][:, :, None], axis=1)
    acc += jnp.dot(u_sel, w1u[...], preferred_element_type=f32)
    acc += jnp.dot(b_e[...], w1b[...], preferred_element_type=f32)
    acc += jnp.dot(a_e[...], w1a[...], preferred_element_type=f32)
    acc += jnp.dot(p_e[...], w1p[...], preferred_element_type=f32)
    acc += jnp.dot(cat[...], w1c[...], preferred_element_type=f32)
    acc += jnp.dot(scal[...], w1s[...], preferred_element_type=f32)
    h1 = jnp.maximum(acc + b1[...], 0.0)
    h2 = jnp.maximum(jnp.dot(h1, w2[...], preferred_element_type=f32) + b2[...], 0.0)
    out = jnp.sum(h2 * w3[...], axis=1, keepdims=True) + b3[0, 0]
    out_ref[...] = out


def _mlp(ft, ub, oh, b_e, a_e, p_e, cat, scal,
         w1t, w1u, w1b, w1a, w1p, w1c, w1s, b1, w2, b2, w3, b3):
    grid = (B // BLK,)
    row = lambda i: (i, 0)
    const = lambda i: (0, 0)
    in_specs = [
        pl.BlockSpec((BLK, F_FT), row),
        pl.BlockSpec((BLK * 16, DU), row),
        pl.BlockSpec((BLK, 16), row),
        pl.BlockSpec((BLK, DP), row),
        pl.BlockSpec((BLK, DP), row),
        pl.BlockSpec((BLK, DP), row),
        pl.BlockSpec((BLK, 9), row),
        pl.BlockSpec((BLK, 4), row),
        pl.BlockSpec((F_FT, 128), const),
        pl.BlockSpec((DU, 128), const),
        pl.BlockSpec((DP, 128), const),
        pl.BlockSpec((DP, 128), const),
        pl.BlockSpec((DP, 128), const),
        pl.BlockSpec((9, 128), const),
        pl.BlockSpec((4, 128), const),
        pl.BlockSpec((1, 128), const),
        pl.BlockSpec((128, 64), const),
        pl.BlockSpec((1, 64), const),
        pl.BlockSpec((1, 64), const),
        pl.BlockSpec((1, 1), const),
    ]
    return pl.pallas_call(
        _mlp_body,
        grid=grid,
        in_specs=in_specs,
        out_specs=pl.BlockSpec((BLK, 1), row),
        out_shape=jax.ShapeDtypeStruct((B, 1), jnp.float32),
    )(ft, ub, oh, b_e, a_e, p_e, cat, scal,
      w1t, w1u, w1b, w1a, w1p, w1c, w1s, b1, w2, b2, w3, b3)


def _prep_idx(ids):
    x = ids.astype(jnp.int32).reshape(NW, NCHUNK, CHUNK)
    x = jnp.pad(x, ((0, 0), (0, 8 - NCHUNK), (0, 0)))
    return x.reshape(NW * 8, CHUNK)


def kernel(user_id, book_id, author_label, category_label, publisher_label,
           page_count, average_rating, ratings_count, published_year,
           full_text_embeddings, user_table, book_table, author_table,
           publisher_table, W1, b1, W2, b2, W3, b3):
    user16 = jnp.pad(user_table, ((0, 0), (0, DU - D))).astype(jnp.bfloat16)
    ub = _sc_gather_user(user16, _prep_idx(user_id))
    oh = jax.nn.one_hot(jnp.remainder(user_id, 16), 16, dtype=jnp.float32)
    b_e = _sc_gather_one(book_table, _prep_idx(book_id))
    a_e = _sc_gather_one(author_table, _prep_idx(author_label))
    p_e = _sc_gather_one(publisher_table, _prep_idx(publisher_label))

    scal = jnp.stack([page_count, average_rating, ratings_count,
                      published_year], axis=1)

    W1T = W1.T

    def padw(w, n):
        return jnp.zeros((n, 128), jnp.float32).at[0:D].set(w)

    w1u = padw(W1T[0:50], DU)
    w1b = padw(W1T[50:100], DP)
    w1a = padw(W1T[100:150], DP)
    w1c = W1T[150:159]
    w1p = padw(W1T[159:209], DP)
    w1s = W1T[209:213]
    w1t = W1T[213:981]

    out = _mlp(full_text_embeddings, ub, oh, b_e, a_e, p_e,
               category_label, scal,
               w1t, w1u, w1b, w1a, w1p, w1c, w1s,
               b1.reshape(1, 128), W2.T, b2.reshape(1, 64),
               W3.reshape(1, 64), b3.reshape(1, 1))
    return out.reshape(B)


# speedup vs baseline: 1.0553x; 1.0553x over previous
import functools

import jax
import jax.numpy as jnp
from jax import lax
from jax.experimental import pallas as pl
from jax.experimental.pallas import tpu as pltpu
from jax.experimental.pallas import tpu_sc as plsc

B = 16384
D = 50
DP = 128
DU = 50
NW = 32
ROWS_PER_W = B // NW
CHUNK = 128
NCHUNK = ROWS_PER_W // CHUNK
HALF = ROWS_PER_W // 2

BLK = 1024
F_FT = 768


def _sel(idx_v, kk):
    a = kk // CHUNK
    col = (kk % CHUNK) // 16 * 16
    iv = idx_v[a, pl.ds(col, 16)]
    lanes = lax.iota(jnp.int32, 16)
    dn = lax.GatherDimensionNumbers(
        offset_dims=(), collapsed_slice_dims=(0,), start_index_map=(0,))
    rot = jnp.bitwise_and(lanes + kk % 16, 15)
    sel = lax.gather(iv, rot.reshape(16, 1), dn, slice_sizes=(1,),
                     mode=lax.GatherScatterMode.PROMISE_IN_BOUNDS)
    return sel[0]


def _sc_gather_body(tab, idx_hbm, out, idx_v, g_v, rows_v, sem):
    c = lax.axis_index("c")
    s = lax.axis_index("s")
    wid = s * 2 + c
    base = wid * ROWS_PER_W
    pltpu.sync_copy(idx_hbm.at[pl.ds(pl.multiple_of(wid * 8, 8), 8)], idx_v)

    zeros16 = jnp.zeros((16,), jnp.float32)

    def zrow(r, _):
        for col in (50, 66, 82, 98, 112):
            rows_v[r, pl.ds(col, 16)] = zeros16
        return _

    lax.fori_loop(0, HALF, zrow, 0)

    for h in range(2):
        def issue(k, _, h=h):
            i = _sel(idx_v, h * HALF + k)
            pltpu.async_copy(tab.at[pl.ds(i, 1)], g_v.at[pl.ds(k, 1)], sem)
            return _

        lax.fori_loop(0, HALF, issue, 0)
        pltpu.make_async_copy(tab.at[pl.ds(0, HALF)], g_v, sem).wait()

        def repack(k, _):
            for o in (0, 16, 32, 34):
                rows_v[k, pl.ds(o, 16)] = g_v[k, pl.ds(o, 16)]
            return _

        lax.fori_loop(0, HALF, repack, 0)
        pltpu.sync_copy(
            rows_v,
            out.at[pl.ds(pl.multiple_of(base + h * HALF, 8), HALF)])


def _sc_gather_one(table, idx):
    mesh = plsc.VectorSubcoreMesh(core_axis_name="c", subcore_axis_name="s")
    return pl.kernel(
        _sc_gather_body,
        out_type=jax.ShapeDtypeStruct((B, DP), jnp.float32),
        mesh=mesh,
        scratch_types=[
            pltpu.VMEM((8, CHUNK), jnp.int32),
            pltpu.VMEM((HALF, D), jnp.float32),
            pltpu.VMEM((HALF, DP), jnp.float32),
            pltpu.SemaphoreType.DMA,
        ],
        compiler_params=pltpu.CompilerParams(use_tc_tiling_on_sc=True),
    )(table, idx)


def _sc_user_body(tab, idx_hbm, out, idx_v, sem):
    c = lax.axis_index("c")
    s = lax.axis_index("s")
    wid = s * 2 + c
    base = wid * ROWS_PER_W
    pltpu.sync_copy(idx_hbm.at[pl.ds(pl.multiple_of(wid * 8, 8), 8)], idx_v)

    def issue(k, _):
        i = _sel(idx_v, k)
        i16 = pl.multiple_of(jnp.bitwise_and(i, ~15), 16)
        pltpu.async_copy(
            tab.at[pl.ds(i16, 16)],
            out.at[pl.ds(pl.multiple_of((base + k) * 16, 16), 16)], sem)
        return _

    lax.fori_loop(0, ROWS_PER_W, issue, 0)
    pltpu.make_async_copy(
        tab.at[pl.ds(0, ROWS_PER_W * 16)],
        out.at[pl.ds(pl.multiple_of(base * 16, 16), ROWS_PER_W * 16)],
        sem).wait()


def _sc_gather_user(table16, idx):
    mesh = plsc.VectorSubcoreMesh(core_axis_name="c", subcore_axis_name="s")
    return pl.kernel(
        _sc_user_body,
        out_type=jax.ShapeDtypeStruct((B * 16, DU), jnp.bfloat16),
        mesh=mesh,
        scratch_types=[
            pltpu.VMEM((8, CHUNK), jnp.int32),
            pltpu.SemaphoreType.DMA,
        ],
        compiler_params=pltpu.CompilerParams(use_tc_tiling_on_sc=True),
    )(table16, idx)


def _mlp_body(ft, ub, oh, b_e, a_e, p_e, cat, scal,
              w1t, w1u, w1b, w1a, w1p, w1c, w1s, b1, w2, b2, w3, b3,
              out_ref):
    f32 = jnp.float32
    acc = jnp.dot(ft[...], w1t[...], preferred_element_type=f32)
    xb = ub[...].astype(f32).reshape(BLK, 16, DU)
    u_sel = jnp.sum(xb * oh[...][:, :, None], axis=1)
    acc += jnp.dot(u_sel, w1u[...], preferred_element_type=f32)
    acc += jnp.dot(b_e[...], w1b[...], preferred_element_type=f32)
    acc += jnp.dot(a_e[...], w1a[...], preferred_element_type=f32)
    acc += jnp.dot(p_e[...], w1p[...], preferred_element_type=f32)
    acc += jnp.dot(cat[...], w1c[...], preferred_element_type=f32)
    acc += jnp.dot(scal[...], w1s[...], preferred_element_type=f32)
    h1 = jnp.maximum(acc + b1[...], 0.0)
    h2 = jnp.maximum(jnp.dot(h1, w2[...], preferred_element_type=f32) + b2[...], 0.0)
    out = jnp.sum(h2 * w3[...], axis=1, keepdims=True) + b3[0, 0]
    out_ref[...] = out


def _mlp(ft, ub, oh, b_e, a_e, p_e, cat, scal,
         w1t, w1u, w1b, w1a, w1p, w1c, w1s, b1, w2, b2, w3, b3):
    grid = (B // BLK,)
    row = lambda i: (i, 0)
    const = lambda i: (0, 0)
    in_specs = [
        pl.BlockSpec((BLK, F_FT), row),
        pl.BlockSpec((BLK * 16, DU), row),
        pl.BlockSpec((BLK, 16), row),
        pl.BlockSpec((BLK, DP), row),
        pl.BlockSpec((BLK, DP), row),
        pl.BlockSpec((BLK, DP), row),
        pl.BlockSpec((BLK, 9), row),
        pl.BlockSpec((BLK, 4), row),
        pl.BlockSpec((F_FT, 128), const),
        pl.BlockSpec((DU, 128), const),
        pl.BlockSpec((DP, 128), const),
        pl.BlockSpec((DP, 128), const),
        pl.BlockSpec((DP, 128), const),
        pl.BlockSpec((9, 128), const),
        pl.BlockSpec((4, 128), const),
        pl.BlockSpec((1, 128), const),
        pl.BlockSpec((128, 64), const),
        pl.BlockSpec((1, 64), const),
        pl.BlockSpec((1, 64), const),
        pl.BlockSpec((1, 1), const),
    ]
    return pl.pallas_call(
        _mlp_body,
        grid=grid,
        in_specs=in_specs,
        out_specs=pl.BlockSpec((BLK, 1), row),
        out_shape=jax.ShapeDtypeStruct((B, 1), jnp.float32),
    )(ft, ub, oh, b_e, a_e, p_e, cat, scal,
      w1t, w1u, w1b, w1a, w1p, w1c, w1s, b1, w2, b2, w3, b3)


def _prep_idx(ids):
    x = ids.astype(jnp.int32).reshape(NW, NCHUNK, CHUNK)
    x = jnp.pad(x, ((0, 0), (0, 8 - NCHUNK), (0, 0)))
    return x.reshape(NW * 8, CHUNK)


def kernel(user_id, book_id, author_label, category_label, publisher_label,
           page_count, average_rating, ratings_count, published_year,
           full_text_embeddings, user_table, book_table, author_table,
           publisher_table, W1, b1, W2, b2, W3, b3):
    user16 = jnp.maximum(user_table, -3.0e38).astype(jnp.bfloat16)
    ub = _sc_gather_user(user16, _prep_idx(user_id))
    oh = jax.nn.one_hot(jnp.remainder(user_id, 16), 16, dtype=jnp.float32)
    b_e = _sc_gather_one(book_table, _prep_idx(book_id))
    a_e = _sc_gather_one(author_table, _prep_idx(author_label))
    p_e = _sc_gather_one(publisher_table, _prep_idx(publisher_label))

    scal = jnp.stack([page_count, average_rating, ratings_count,
                      published_year], axis=1)

    W1T = W1.T

    def padw(w, n):
        return jnp.zeros((n, 128), jnp.float32).at[0:D].set(w)

    w1u = W1T[0:50]
    w1b = padw(W1T[50:100], DP)
    w1a = padw(W1T[100:150], DP)
    w1c = W1T[150:159]
    w1p = padw(W1T[159:209], DP)
    w1s = W1T[209:213]
    w1t = W1T[213:981]

    out = _mlp(full_text_embeddings, ub, oh, b_e, a_e, p_e,
               category_label, scal,
               w1t, w1u, w1b, w1a, w1p, w1c, w1s,
               b1.reshape(1, 128), W2.T, b2.reshape(1, 64),
               W3.reshape(1, 64), b3.reshape(1, 1))
    return out.reshape(B)


# speedup vs baseline: 4.0091x; 3.7990x over previous
import functools

import jax
import jax.numpy as jnp
from jax import lax
from jax.experimental import pallas as pl
from jax.experimental.pallas import tpu as pltpu
from jax.experimental.pallas import tpu_sc as plsc

B = 16384
D = 50
DP = 128
DU = 50
NW = 32
ROWS_PER_W = B // NW
CHUNK = 128
NCHUNK = ROWS_PER_W // CHUNK
HALF = ROWS_PER_W // 2

BLK = 1024
F_FT = 768


def _sel(idx_v, kk):
    a = kk // CHUNK
    col = (kk % CHUNK) // 16 * 16
    iv = idx_v[a, pl.ds(col, 16)]
    lanes = lax.iota(jnp.int32, 16)
    dn = lax.GatherDimensionNumbers(
        offset_dims=(), collapsed_slice_dims=(0,), start_index_map=(0,))
    rot = jnp.bitwise_and(lanes + kk % 16, 15)
    sel = lax.gather(iv, rot.reshape(16, 1), dn, slice_sizes=(1,),
                     mode=lax.GatherScatterMode.PROMISE_IN_BOUNDS)
    return sel[0]


def _sc_gather_body(tab, idx_hbm, out, idx_v, g_v, rows_v, sem):
    c = lax.axis_index("c")
    s = lax.axis_index("s")
    wid = s * 2 + c
    base = wid * ROWS_PER_W
    pltpu.sync_copy(idx_hbm.at[pl.ds(pl.multiple_of(wid * 8, 8), 8)], idx_v)

    zeros16 = jnp.zeros((16,), jnp.float32)

    def zrow(r, _):
        for col in (50, 66, 82, 98, 112):
            rows_v[r, pl.ds(col, 16)] = zeros16
        return _

    lax.fori_loop(0, HALF, zrow, 0)

    for h in range(2):
        def issue(k, _, h=h):
            i = _sel(idx_v, h * HALF + k)
            pltpu.async_copy(tab.at[pl.ds(i, 1)], g_v.at[pl.ds(k, 1)], sem)
            return _

        lax.fori_loop(0, HALF, issue, 0)
        pltpu.make_async_copy(tab.at[pl.ds(0, HALF)], g_v, sem).wait()

        def repack(k, _):
            for o in (0, 16, 32, 34):
                rows_v[k, pl.ds(o, 16)] = g_v[k, pl.ds(o, 16)]
            return _

        lax.fori_loop(0, HALF, repack, 0)
        pltpu.sync_copy(
            rows_v,
            out.at[pl.ds(pl.multiple_of(base + h * HALF, 8), HALF)])


def _sc_gather_one(table, idx):
    mesh = plsc.VectorSubcoreMesh(core_axis_name="c", subcore_axis_name="s")
    return pl.kernel(
        _sc_gather_body,
        out_type=jax.ShapeDtypeStruct((B, DP), jnp.float32),
        mesh=mesh,
        scratch_types=[
            pltpu.VMEM((8, CHUNK), jnp.int32),
            pltpu.VMEM((HALF, D), jnp.float32),
            pltpu.VMEM((HALF, DP), jnp.float32),
            pltpu.SemaphoreType.DMA,
        ],
        compiler_params=pltpu.CompilerParams(use_tc_tiling_on_sc=True),
    )(table, idx)


GCH = 64


def _sc_user_body(tab, idx_hbm, out, idx_v, g_v, sem):
    c = lax.axis_index("c")
    s = lax.axis_index("s")
    wid = s * 2 + c
    base = wid * ROWS_PER_W
    pltpu.sync_copy(idx_hbm.at[pl.ds(pl.multiple_of(wid * 8, 8), 8)], idx_v)

    for p in range(ROWS_PER_W // GCH):
        def issue(k, _, p=p):
            i = _sel(idx_v, p * GCH + k)
            i16 = pl.multiple_of(jnp.bitwise_and(i, ~15), 16)
            pltpu.async_copy(
                tab.at[pl.ds(i16, 16)],
                g_v.at[pl.ds(pl.multiple_of(k * 16, 16), 16)], sem)
            return _

        lax.fori_loop(0, GCH, issue, 0)
        pltpu.make_async_copy(tab.at[pl.ds(0, GCH * 16)], g_v, sem).wait()
        pltpu.sync_copy(
            g_v,
            out.at[pl.ds(pl.multiple_of((base + p * GCH) * 16, 1024),
                         GCH * 16)])


def _sc_gather_user(table16, idx):
    mesh = plsc.VectorSubcoreMesh(core_axis_name="c", subcore_axis_name="s")
    return pl.kernel(
        _sc_user_body,
        out_type=jax.ShapeDtypeStruct((B * 16, DU), jnp.bfloat16),
        mesh=mesh,
        scratch_types=[
            pltpu.VMEM((8, CHUNK), jnp.int32),
            pltpu.VMEM((GCH * 16, DU), jnp.bfloat16),
            pltpu.SemaphoreType.DMA,
        ],
        compiler_params=pltpu.CompilerParams(use_tc_tiling_on_sc=True),
    )(table16, idx)


def _mlp_body(ft, ub, oh, b_e, a_e, p_e, cat, scal,
              w1t, w1u, w1b, w1a, w1p, w1c, w1s, b1, w2, b2, w3, b3,
              out_ref):
    f32 = jnp.float32
    acc = jnp.dot(ft[...], w1t[...], preferred_element_type=f32)
    xb = ub[...].astype(f32).reshape(BLK, 16, DU)
    u_sel = jnp.sum(xb * oh[...][:, :, None], axis=1)
    acc += jnp.dot(u_sel, w1u[...], preferred_element_type=f32)
    acc += jnp.dot(b_e[...], w1b[...], preferred_element_type=f32)
    acc += jnp.dot(a_e[...], w1a[...], preferred_element_type=f32)
    acc += jnp.dot(p_e[...], w1p[...], preferred_element_type=f32)
    acc += jnp.dot(cat[...], w1c[...], preferred_element_type=f32)
    acc += jnp.dot(scal[...], w1s[...], preferred_element_type=f32)
    h1 = jnp.maximum(acc + b1[...], 0.0)
    h2 = jnp.maximum(jnp.dot(h1, w2[...], preferred_element_type=f32) + b2[...], 0.0)
    out = jnp.sum(h2 * w3[...], axis=1, keepdims=True) + b3[0, 0]
    out_ref[...] = out


def _mlp(ft, ub, oh, b_e, a_e, p_e, cat, scal,
         w1t, w1u, w1b, w1a, w1p, w1c, w1s, b1, w2, b2, w3, b3):
    grid = (B // BLK,)
    row = lambda i: (i, 0)
    const = lambda i: (0, 0)
    in_specs = [
        pl.BlockSpec((BLK, F_FT), row),
        pl.BlockSpec((BLK * 16, DU), row),
        pl.BlockSpec((BLK, 16), row),
        pl.BlockSpec((BLK, DP), row),
        pl.BlockSpec((BLK, DP), row),
        pl.BlockSpec((BLK, DP), row),
        pl.BlockSpec((BLK, 9), row),
        pl.BlockSpec((BLK, 4), row),
        pl.BlockSpec((F_FT, 128), const),
        pl.BlockSpec((DU, 128), const),
        pl.BlockSpec((DP, 128), const),
        pl.BlockSpec((DP, 128), const),
        pl.BlockSpec((DP, 128), const),
        pl.BlockSpec((9, 128), const),
        pl.BlockSpec((4, 128), const),
        pl.BlockSpec((1, 128), const),
        pl.BlockSpec((128, 64), const),
        pl.BlockSpec((1, 64), const),
        pl.BlockSpec((1, 64), const),
        pl.BlockSpec((1, 1), const),
    ]
    return pl.pallas_call(
        _mlp_body,
        grid=grid,
        in_specs=in_specs,
        out_specs=pl.BlockSpec((BLK, 1), row),
        out_shape=jax.ShapeDtypeStruct((B, 1), jnp.float32),
    )(ft, ub, oh, b_e, a_e, p_e, cat, scal,
      w1t, w1u, w1b, w1a, w1p, w1c, w1s, b1, w2, b2, w3, b3)


def _prep_idx(ids):
    x = ids.astype(jnp.int32).reshape(NW, NCHUNK, CHUNK)
    x = jnp.pad(x, ((0, 0), (0, 8 - NCHUNK), (0, 0)))
    return x.reshape(NW * 8, CHUNK)


def kernel(user_id, book_id, author_label, category_label, publisher_label,
           page_count, average_rating, ratings_count, published_year,
           full_text_embeddings, user_table, book_table, author_table,
           publisher_table, W1, b1, W2, b2, W3, b3):
    user16 = jnp.maximum(user_table, -3.0e38).astype(jnp.bfloat16)
    ub = _sc_gather_user(user16, _prep_idx(user_id))
    oh = jax.nn.one_hot(jnp.remainder(user_id, 16), 16, dtype=jnp.float32)
    b_e = _sc_gather_one(book_table, _prep_idx(book_id))
    a_e = _sc_gather_one(author_table, _prep_idx(author_label))
    p_e = _sc_gather_one(publisher_table, _prep_idx(publisher_label))

    scal = jnp.stack([page_count, average_rating, ratings_count,
                      published_year], axis=1)

    W1T = W1.T

    def padw(w, n):
        return jnp.zeros((n, 128), jnp.float32).at[0:D].set(w)

    w1u = W1T[0:50]
    w1b = padw(W1T[50:100], DP)
    w1a = padw(W1T[100:150], DP)
    w1c = W1T[150:159]
    w1p = padw(W1T[159:209], DP)
    w1s = W1T[209:213]
    w1t = W1T[213:981]

    out = _mlp(full_text_embeddings, ub, oh, b_e, a_e, p_e,
               category_label, scal,
               w1t, w1u, w1b, w1a, w1p, w1c, w1s,
               b1.reshape(1, 128), W2.T, b2.reshape(1, 64),
               W3.reshape(1, 64), b3.reshape(1, 1))
    return out.reshape(B)


# speedup vs baseline: 5.2338x; 1.3055x over previous
import functools

import jax
import jax.numpy as jnp
from jax import lax
from jax.experimental import pallas as pl
from jax.experimental.pallas import tpu as pltpu
from jax.experimental.pallas import tpu_sc as plsc

B = 16384
D = 50
DP = 128
NW = 32
ROWS_PER_W = B // NW
CHUNK = 128
NCHUNK = ROWS_PER_W // CHUNK
HALF = ROWS_PER_W // 2

BLK = 1024
F_FT = 768


def _sc_gather_body(tab, idx_hbm, out, idx_v, g_v, rows_v, sem):
    c = lax.axis_index("c")
    s = lax.axis_index("s")
    wid = s * 2 + c
    base = wid * ROWS_PER_W
    pltpu.sync_copy(idx_hbm.at[pl.ds(pl.multiple_of(wid * 8, 8), 8)], idx_v)

    zeros16 = jnp.zeros((16,), jnp.float32)

    def zrow(r, _):
        for col in (50, 66, 82, 98, 112):
            rows_v[r, pl.ds(col, 16)] = zeros16
        return _

    lax.fori_loop(0, HALF, zrow, 0)

    lanes = lax.iota(jnp.int32, 16)
    for h in range(2):
        def issue(k, _, h=h):
            kk = h * HALF + k
            a = kk // CHUNK
            col = (kk % CHUNK) // 16 * 16
            iv = idx_v[a, pl.ds(col, 16)]
            dn = lax.GatherDimensionNumbers(
                offset_dims=(), collapsed_slice_dims=(0,),
                start_index_map=(0,))
            rot = jnp.bitwise_and(lanes + kk % 16, 15)
            sel = lax.gather(iv, rot.reshape(16, 1), dn,
                             slice_sizes=(1,),
                             mode=lax.GatherScatterMode.PROMISE_IN_BOUNDS)
            i = sel[0]
            pltpu.async_copy(tab.at[pl.ds(i, 1)], g_v.at[pl.ds(k, 1)], sem)
            return _

        lax.fori_loop(0, HALF, issue, 0)
        pltpu.make_async_copy(tab.at[pl.ds(0, HALF)], g_v, sem).wait()

        def repack(k, _):
            for o in (0, 16, 32, 34):
                rows_v[k, pl.ds(o, 16)] = g_v[k, pl.ds(o, 16)]
            return _

        lax.fori_loop(0, HALF, repack, 0)
        pltpu.sync_copy(
            rows_v,
            out.at[pl.ds(pl.multiple_of(base + h * HALF, 8), HALF)])


def _sc_gather_one(table, idx):
    mesh = plsc.VectorSubcoreMesh(core_axis_name="c", subcore_axis_name="s")
    return pl.kernel(
        _sc_gather_body,
        out_type=jax.ShapeDtypeStruct((B, DP), jnp.float32),
        mesh=mesh,
        scratch_types=[
            pltpu.VMEM((8, CHUNK), jnp.int32),
            pltpu.VMEM((HALF, D), jnp.float32),
            pltpu.VMEM((HALF, DP), jnp.float32),
            pltpu.SemaphoreType.DMA,
        ],
        compiler_params=pltpu.CompilerParams(use_tc_tiling_on_sc=True),
    )(table, idx)


def _mlp_body(ft, u_e, b_e, a_e, p_e, cat, scal,
              w1t, w1u, w1b, w1a, w1p, w1c, w1s, b1, w2, b2, w3, b3,
              out_ref):
    f32 = jnp.float32
    acc = jnp.dot(ft[...], w1t[...], preferred_element_type=f32)
    acc += jnp.dot(u_e[...], w1u[...], preferred_element_type=f32)
    acc += jnp.dot(b_e[...], w1b[...], preferred_element_type=f32)
    acc += jnp.dot(a_e[...], w1a[...], preferred_element_type=f32)
    acc += jnp.dot(p_e[...], w1p[...], preferred_element_type=f32)
    acc += jnp.dot(cat[...], w1c[...], preferred_element_type=f32)
    acc += jnp.dot(scal[...], w1s[...], preferred_element_type=f32)
    h1 = jnp.maximum(acc + b1[...], 0.0)
    h2 = jnp.maximum(jnp.dot(h1, w2[...], preferred_element_type=f32) + b2[...], 0.0)
    out = jnp.sum(h2 * w3[...], axis=1, keepdims=True) + b3[0, 0]
    out_ref[...] = out


def _mlp(ft, u_e, b_e, a_e, p_e, cat, scal,
         w1t, w1u, w1b, w1a, w1p, w1c, w1s, b1, w2, b2, w3, b3):
    grid = (B // BLK,)
    row = lambda i: (i, 0)
    const = lambda i: (0, 0)
    in_specs = [
        pl.BlockSpec((BLK, F_FT), row),
        pl.BlockSpec((BLK, DP), row),
        pl.BlockSpec((BLK, DP), row),
        pl.BlockSpec((BLK, DP), row),
        pl.BlockSpec((BLK, DP), row),
        pl.BlockSpec((BLK, 9), row),
        pl.BlockSpec((BLK, 4), row),
        pl.BlockSpec((F_FT, 128), const),
        pl.BlockSpec((DP, 128), const),
        pl.BlockSpec((DP, 128), const),
        pl.BlockSpec((DP, 128), const),
        pl.BlockSpec((DP, 128), const),
        pl.BlockSpec((9, 128), const),
        pl.BlockSpec((4, 128), const),
        pl.BlockSpec((1, 128), const),
        pl.BlockSpec((128, 64), const),
        pl.BlockSpec((1, 64), const),
        pl.BlockSpec((1, 64), const),
        pl.BlockSpec((1, 1), const),
    ]
    return pl.pallas_call(
        _mlp_body,
        grid=grid,
        in_specs=in_specs,
        out_specs=pl.BlockSpec((BLK, 1), row),
        out_shape=jax.ShapeDtypeStruct((B, 1), jnp.float32),
    )(ft, u_e, b_e, a_e, p_e, cat, scal,
      w1t, w1u, w1b, w1a, w1p, w1c, w1s, b1, w2, b2, w3, b3)


def _prep_idx(ids):
    x = ids.astype(jnp.int32).reshape(NW, NCHUNK, CHUNK)
    x = jnp.pad(x, ((0, 0), (0, 8 - NCHUNK), (0, 0)))
    return x.reshape(NW * 8, CHUNK)


def kernel(user_id, book_id, author_label, category_label, publisher_label,
           page_count, average_rating, ratings_count, published_year,
           full_text_embeddings, user_table, book_table, author_table,
           publisher_table, W1, b1, W2, b2, W3, b3):
    u_e = _sc_gather_one(user_table, _prep_idx(user_id))
    b_e = _sc_gather_one(book_table, _prep_idx(book_id))
    a_e = _sc_gather_one(author_table, _prep_idx(author_label))
    p_e = _sc_gather_one(publisher_table, _prep_idx(publisher_label))

    scal = jnp.stack([page_count, average_rating, ratings_count,
                      published_year], axis=1)

    W1T = W1.T

    def padw(w):
        return jnp.zeros((DP, 128), jnp.float32).at[0:D].set(w)

    w1u = padw(W1T[0:50])
    w1b = padw(W1T[50:100])
    w1a = padw(W1T[100:150])
    w1c = W1T[150:159]
    w1p = padw(W1T[159:209])
    w1s = W1T[209:213]
    w1t = W1T[213:981]

    out = _mlp(full_text_embeddings, u_e, b_e, a_e, p_e,
               category_label, scal,
               w1t, w1u, w1b, w1a, w1p, w1c, w1s,
               b1.reshape(1, 128), W2.T, b2.reshape(1, 64),
               W3.reshape(1, 64), b3.reshape(1, 1))
    return out.reshape(B)
